# baseline (speedup 1.0000x reference)
import functools
import jax
import jax.numpy as jnp
from jax import lax
from jax.experimental import pallas as pl
from jax.experimental.pallas import tpu as pltpu
from jax.experimental.pallas import tpu_sc as plsc

N_CLS_CTX = 4
CTX_DIM = 512
SEQ_LEN = 77
N_PRE = N_CLS_CTX + 1
N_SUF = SEQ_LEN - 2 * N_CLS_CTX - 1
GRP = 8
LANES = 16


def kernel(vehicle_ids, tokenized_prompts, token_embedding, cls_ctx):
    B = tokenized_prompts.shape[0]
    info = plsc.get_sparse_core_info()
    nc, ns = info.num_cores, info.num_subcores
    nw = nc * ns
    n_per_w = B // nw

    tp = tokenized_prompts.astype(jnp.int32)
    tp_pre = tp[:, :N_PRE].reshape(-1)
    tp_suf = tp[:, N_PRE + N_CLS_CTX:]
    cls2d = cls_ctx.reshape(cls_ctx.shape[0] * N_CLS_CTX, CTX_DIM)
    vid4 = (vehicle_ids.astype(jnp.int32)[:, None] * N_CLS_CTX
            + jnp.arange(N_CLS_CTX, dtype=jnp.int32)[None, :]).reshape(-1)

    mesh = plsc.VectorSubcoreMesh(core_axis_name="c", subcore_axis_name="s")

    @functools.partial(
        pl.kernel,
        mesh=mesh,
        compiler_params=pltpu.CompilerParams(use_tc_tiling_on_sc=False),
        out_type=jax.ShapeDtypeStruct((B, SEQ_LEN, CTX_DIM), jnp.float32),
        scratch_types=[
            pltpu.VMEM((n_per_w * N_PRE,), jnp.int32),
            pltpu.VMEM((n_per_w, N_SUF), jnp.int32),
            pltpu.VMEM((n_per_w * N_CLS_CTX,), jnp.int32),
            pltpu.VMEM((GRP * N_PRE, CTX_DIM), jnp.float32),
            pltpu.VMEM((GRP * N_CLS_CTX, CTX_DIM), jnp.float32),
            pltpu.VMEM((SEQ_LEN, CTX_DIM), jnp.float32),
            pltpu.VMEM((SEQ_LEN, CTX_DIM), jnp.float32),
            pltpu.SemaphoreType.DMA,
            pltpu.SemaphoreType.DMA,
            pltpu.SemaphoreType.DMA,
            pltpu.SemaphoreType.DMA,
        ],
    )
    def prompt_gather(pre_hbm, suf_hbm, vid4_hbm, te_hbm, cls_hbm, out_hbm,
                      pre_v, suf_v, vid4_v, stg_pre, stg_cls, rows0, rows1,
                      gsem, psem, ssem0, ssem1):
        wid = lax.axis_index("s") * nc + lax.axis_index("c")
        base = wid * n_per_w
        pltpu.sync_copy(pre_hbm.at[pl.ds(base * N_PRE, n_per_w * N_PRE)], pre_v)
        pltpu.sync_copy(suf_hbm.at[pl.ds(base, n_per_w), :], suf_v)
        pltpu.sync_copy(vid4_hbm.at[pl.ds(base * N_CLS_CTX, n_per_w * N_CLS_CTX)],
                        vid4_v)

        def place(rows_v, j):
            for r in range(N_PRE):
                for c in range(CTX_DIM // LANES):
                    rows_v[r, pl.ds(c * LANES, LANES)] = (
                        stg_pre[j * N_PRE + r, pl.ds(c * LANES, LANES)])
            for r in range(N_CLS_CTX):
                for c in range(CTX_DIM // LANES):
                    rows_v[N_PRE + r, pl.ds(c * LANES, LANES)] = (
                        stg_cls[j * N_CLS_CTX + r, pl.ds(c * LANES, LANES)])

        def one_item(g, j, rows_v, ssem, stg_waits):
            i = GRP * g + j
            b = base + i
            if j < 2:
                @pl.when(g > 0)
                def _():
                    pltpu.make_async_copy(rows_v, out_hbm.at[b], ssem).wait()
            else:
                pltpu.make_async_copy(rows_v, out_hbm.at[b], ssem).wait()
            gs = pltpu.async_copy(te_hbm.at[suf_v.at[i]],
                                  rows_v.at[pl.ds(N_PRE + N_CLS_CTX, N_SUF)],
                                  gsem)
            for w in stg_waits:
                w.wait()
            gs.wait()
            place(rows_v, j)
            pltpu.async_copy(rows_v, out_hbm.at[b], ssem)

        def body(g, carry):
            gp = pltpu.async_copy(
                te_hbm.at[pre_v.at[pl.ds(g * (GRP * N_PRE), GRP * N_PRE)]],
                stg_pre, psem)
            gc = pltpu.async_copy(
                cls_hbm.at[vid4_v.at[pl.ds(g * (GRP * N_CLS_CTX),
                                           GRP * N_CLS_CTX)]],
                stg_cls, psem)
            one_item(g, 0, rows0, ssem0, (gp, gc))
            one_item(g, 1, rows1, ssem1, ())
            one_item(g, 2, rows0, ssem0, ())
            one_item(g, 3, rows1, ssem1, ())
            one_item(g, 4, rows0, ssem0, ())
            one_item(g, 5, rows1, ssem1, ())
            one_item(g, 6, rows0, ssem0, ())
            one_item(g, 7, rows1, ssem1, ())
            return carry

        lax.fori_loop(0, n_per_w // GRP, body, 0)
        pltpu.make_async_copy(rows0, out_hbm.at[base], ssem0).wait()
        pltpu.make_async_copy(rows1, out_hbm.at[base], ssem1).wait()

    return prompt_gather(tp_pre, tp_suf, vid4, token_embedding, cls2d)


# speedup vs baseline: 1.0083x; 1.0083x over previous
import functools
import jax
import jax.numpy as jnp
from jax import lax
from jax.experimental import pallas as pl
from jax.experimental.pallas import tpu as pltpu
from jax.experimental.pallas import tpu_sc as plsc

N_CLS_CTX = 4
CTX_DIM = 512
SEQ_LEN = 77
N_PRE = N_CLS_CTX + 1
N_SUF = SEQ_LEN - 2 * N_CLS_CTX - 1


def kernel(vehicle_ids, tokenized_prompts, token_embedding, cls_ctx):
    B = tokenized_prompts.shape[0]
    info = plsc.get_sparse_core_info()
    nc, ns = info.num_cores, info.num_subcores
    nw = nc * ns
    n_per_w = B // nw

    tp = tokenized_prompts.astype(jnp.int32)
    tp_pre = tp[:, :N_PRE]
    tp_suf = tp[:, N_PRE + N_CLS_CTX:]
    cls2d = cls_ctx.reshape(cls_ctx.shape[0] * N_CLS_CTX, CTX_DIM)
    vid4 = (vehicle_ids.astype(jnp.int32)[:, None] * N_CLS_CTX
            + jnp.arange(N_CLS_CTX, dtype=jnp.int32)[None, :])

    mesh = plsc.VectorSubcoreMesh(core_axis_name="c", subcore_axis_name="s")

    @functools.partial(
        pl.kernel,
        mesh=mesh,
        compiler_params=pltpu.CompilerParams(use_tc_tiling_on_sc=False),
        out_type=jax.ShapeDtypeStruct((B, SEQ_LEN, CTX_DIM), jnp.float32),
        scratch_types=[
            pltpu.VMEM((n_per_w, N_PRE), jnp.int32),
            pltpu.VMEM((n_per_w, N_SUF), jnp.int32),
            pltpu.VMEM((n_per_w, N_CLS_CTX), jnp.int32),
            pltpu.VMEM((SEQ_LEN, CTX_DIM), jnp.float32),
            pltpu.VMEM((SEQ_LEN, CTX_DIM), jnp.float32),
            pltpu.VMEM((SEQ_LEN, CTX_DIM), jnp.float32),
            pltpu.SemaphoreType.DMA,
            pltpu.SemaphoreType.DMA,
            pltpu.SemaphoreType.DMA,
            pltpu.SemaphoreType.DMA,
        ],
    )
    def prompt_gather(pre_hbm, suf_hbm, vid4_hbm, te_hbm, cls_hbm, out_hbm,
                      pre_v, suf_v, vid4_v, rows0, rows1, rows2,
                      gsem, ssem0, ssem1, ssem2):
        wid = lax.axis_index("s") * nc + lax.axis_index("c")
        base = wid * n_per_w
        pltpu.sync_copy(pre_hbm.at[pl.ds(base, n_per_w), :], pre_v)
        pltpu.sync_copy(suf_hbm.at[pl.ds(base, n_per_w), :], suf_v)
        pltpu.sync_copy(vid4_hbm.at[pl.ds(base, n_per_w), :], vid4_v)

        def one_item(k, i, rows_v, ssem):
            b = base + i
            @pl.when(k > 0)
            def _():
                pltpu.make_async_copy(rows_v, out_hbm.at[b], ssem).wait()
            g1 = pltpu.async_copy(te_hbm.at[pre_v.at[i]],
                                  rows_v.at[pl.ds(0, N_PRE)], gsem)
            g2 = pltpu.async_copy(cls_hbm.at[vid4_v.at[i]],
                                  rows_v.at[pl.ds(N_PRE, N_CLS_CTX)], gsem)
            g3 = pltpu.async_copy(te_hbm.at[suf_v.at[i]],
                                  rows_v.at[pl.ds(N_PRE + N_CLS_CTX, N_SUF)],
                                  gsem)
            g1.wait()
            g2.wait()
            g3.wait()
            pltpu.async_copy(rows_v, out_hbm.at[b], ssem)

        def body(k, carry):
            one_item(k, 3 * k, rows0, ssem0)
            one_item(k, 3 * k + 1, rows1, ssem1)
            one_item(k, 3 * k + 2, rows2, ssem2)
            return carry

        n_loop = n_per_w // 3
        lax.fori_loop(0, n_loop, body, 0)
        for t, (rv, sv) in enumerate(((rows0, ssem0), (rows1, ssem1))):
            if 3 * n_loop + t < n_per_w:
                one_item(1, 3 * n_loop + t, rv, sv)
        pltpu.make_async_copy(rows0, out_hbm.at[base], ssem0).wait()
        pltpu.make_async_copy(rows1, out_hbm.at[base], ssem1).wait()
        pltpu.make_async_copy(rows2, out_hbm.at[base], ssem2).wait()

    return prompt_gather(tp_pre, tp_suf, vid4, token_embedding, cls2d)
